# Initial kernel scaffold; baseline (speedup 1.0000x reference)
import jax
import jax.numpy as jnp
from jax.experimental import pallas as pl


def kernel(x, weights):
    raise NotImplementedError("write your pallas kernel here")


# speedup vs baseline: 17.4960x; 17.4960x over previous
import functools

import jax
import jax.numpy as jnp
import numpy as np
import ml_dtypes
from jax.experimental import pallas as pl


_L = 3000
_PATCH = 75
_NP_ = _L // _PATCH
_KC = _L // 2 + 1
_KPAD = 1536


def _patch_hits_table(L, patch_size, num_patches):
    freqs = np.fft.fftfreq(L)
    with np.errstate(divide="ignore"):
        periods = np.abs(1.0 / freqs)
    periods[np.isinf(periods)] = 0
    hits = np.zeros((L, num_patches), dtype=bool)
    for k in range(L):
        p = periods[k]
        if p == 0:
            continue
        interval = int(p)
        peaks = np.arange(0, L, interval)
        pidx = np.floor(peaks / patch_size).astype(np.int64)
        pidx = np.unique(pidx[pidx < num_patches])
        hits[k, pidx] = True
    return hits


@functools.lru_cache(maxsize=1)
def _constants():
    t = np.arange(_L, dtype=np.int64)
    k = np.arange(_KPAD, dtype=np.int64)
    ang = (2.0 * np.pi / _L) * ((t[:, None] * k[None, :]) % _L).astype(np.float64)
    d = np.concatenate([np.cos(ang), np.sin(ang)], axis=1).astype(np.float32)
    d_hi = d.astype(ml_dtypes.bfloat16)
    d_lo = (d - d_hi.astype(np.float32)).astype(ml_dtypes.bfloat16)
    hits = _patch_hits_table(_L, _PATCH, _NP_)[:_KC]
    hits_pad = np.zeros((_KPAD, _NP_), dtype=ml_dtypes.bfloat16)
    hits_pad[:_KC] = hits.astype(ml_dtypes.bfloat16)
    return d_hi, d_lo, hits_pad


def _body(x_ref, dh_ref, dl_ref, ht_ref, w_ref, out_ref):
    xb = x_ref[0]
    xh = xb.astype(jnp.bfloat16)
    xl = (xb - xh.astype(jnp.float32)).astype(jnp.bfloat16)
    z = (jnp.dot(xh, dh_ref[...], preferred_element_type=jnp.float32)
         + jnp.dot(xh, dl_ref[...], preferred_element_type=jnp.float32)
         + jnp.dot(xl, dh_ref[...], preferred_element_type=jnp.float32))
    zc = z[:, :_KPAD]
    zs = z[:, _KPAD:]
    e = zc * zc + zs * zs
    kiota = jax.lax.broadcasted_iota(jnp.int32, (128, _KPAD), 1)
    e = jnp.where(kiota < _KC, e, -1.0)

    idxs, mults = [], []
    for _ in range(4):
        mx = jnp.max(e, axis=1, keepdims=True)
        cand = jnp.where(e == mx, kiota, jnp.int32(1 << 20))
        ij = jnp.min(cand, axis=1, keepdims=True)
        e = jnp.where(kiota == ij, -1.0, e)
        idxs.append(ij)
        mults.append(jnp.where((ij == 0) | (ij == _L // 2), 1, 2).astype(jnp.int32))

    c0 = jnp.zeros_like(mults[0])
    c1 = mults[0]
    c2 = c1 + mults[1]
    c3 = c2 + mults[2]
    starts = [c0, c1, c2, c3]

    cnt = jnp.zeros((128, _KPAD), jnp.bfloat16)
    for p in (1, 3, 5):
        pick = sum(
            idxs[j] * ((starts[j] <= p) & (p < starts[j] + mults[j])).astype(jnp.int32)
            for j in range(4)
        )
        cnt = cnt + (kiota == pick).astype(jnp.bfloat16)

    part = jnp.dot(cnt, ht_ref[...], preferred_element_type=jnp.float32)
    out_ref[0] = jnp.sum(part, axis=0, keepdims=True) * w_ref[...]


def kernel(x, weights):
    B, N, L = x.shape
    d_hi, d_lo, hits_pad = _constants()
    w = jnp.reshape(weights, (1, 1)).astype(jnp.float32)
    out_t = pl.pallas_call(
        _body,
        grid=(B,),
        in_specs=[
            pl.BlockSpec((1, N, L), lambda b: (b, 0, 0)),
            pl.BlockSpec((L, 2 * _KPAD), lambda b: (0, 0)),
            pl.BlockSpec((L, 2 * _KPAD), lambda b: (0, 0)),
            pl.BlockSpec((_KPAD, _NP_), lambda b: (0, 0)),
            pl.BlockSpec((1, 1), lambda b: (0, 0)),
        ],
        out_specs=pl.BlockSpec((1, 1, _NP_), lambda b: (b, 0, 0)),
        out_shape=jax.ShapeDtypeStruct((B, 1, _NP_), jnp.float32),
    )(x, jnp.asarray(d_hi), jnp.asarray(d_lo), jnp.asarray(hits_pad), w)
    return out_t[:, 0, :]
